# Initial kernel scaffold; baseline (speedup 1.0000x reference)
import jax
import jax.numpy as jnp
from jax.experimental import pallas as pl


def kernel(x_categorical, tables, W1, b1, Wp, bp):
    raise NotImplementedError("write your pallas kernel here")


# speedup vs baseline: 7.8545x; 7.8545x over previous
import functools

import jax
import jax.numpy as jnp
from jax import lax
from jax.experimental import pallas as pl
from jax.experimental.pallas import tpu as pltpu
from jax.experimental.pallas import tpu_sc as plsc

_NC = 2
_NS = 16
_NW = _NC * _NS
_IDXS_PER_DMA = 128
_DMAS_PER_CHUNK = 8


def _sc_gather(table_flat, idx3, n_rows, d):
    rpw = n_rows // _NW
    n_dma = rpw // _IDXS_PER_DMA
    chunk_rows = _IDXS_PER_DMA * _DMAS_PER_CHUNK
    n_chunks = n_dma // _DMAS_PER_CHUNK
    mesh = plsc.VectorSubcoreMesh(core_axis_name="c", subcore_axis_name="s")

    @functools.partial(
        pl.kernel,
        out_type=jax.ShapeDtypeStruct((n_rows, d), jnp.float32),
        mesh=mesh,
        compiler_params=pltpu.CompilerParams(use_tc_tiling_on_sc=False),
        scratch_types=[
            pltpu.VMEM((n_dma, _IDXS_PER_DMA), jnp.int32),
            pltpu.VMEM((chunk_rows, d), jnp.float32),
            pltpu.SemaphoreType.DMA,
        ],
    )
    def gather_k(table_hbm, idx_hbm, out_hbm, idx_v, rows_v, gsem):
        wid = lax.axis_index("s") * _NC + lax.axis_index("c")
        row0 = wid * rpw
        pltpu.sync_copy(idx_hbm.at[wid], idx_v)

        def chunk_body(c, carry):
            copies = []
            for m in range(_DMAS_PER_CHUNK):
                cp = pltpu.async_copy(
                    table_hbm.at[idx_v.at[c * _DMAS_PER_CHUNK + m]],
                    rows_v.at[pl.ds(m * _IDXS_PER_DMA, _IDXS_PER_DMA)],
                    gsem,
                )
                copies.append(cp)
            for cp in copies:
                cp.wait()
            off = pl.multiple_of(row0 + c * chunk_rows, chunk_rows)
            pltpu.sync_copy(rows_v, out_hbm.at[pl.ds(off, chunk_rows)])
            return carry

        lax.fori_loop(0, n_chunks, chunk_body, 0)

    return gather_k(table_flat, idx3)


def _tc_mlp(x, w1, b1, wp, bp, bt):
    b, d_in = x.shape
    h = w1.shape[1]

    def mlp_k(x_ref, w1_ref, b1_ref, wp_ref, bp_ref, o_ref):
        acc = jnp.dot(x_ref[...], w1_ref[...], preferred_element_type=jnp.float32)
        acc = jnp.maximum(acc + b1_ref[...], 0.0)
        out = jnp.dot(acc, wp_ref[...], preferred_element_type=jnp.float32)
        o_ref[...] = jax.nn.sigmoid(out + bp_ref[...])

    return pl.pallas_call(
        mlp_k,
        grid=(b // bt,),
        in_specs=[
            pl.BlockSpec((bt, d_in), lambda i: (i, 0)),
            pl.BlockSpec((d_in, h), lambda i: (0, 0)),
            pl.BlockSpec((1, h), lambda i: (0, 0)),
            pl.BlockSpec((h, 1), lambda i: (0, 0)),
            pl.BlockSpec((1, 1), lambda i: (0, 0)),
        ],
        out_specs=pl.BlockSpec((bt, 1), lambda i: (i, 0)),
        out_shape=jax.ShapeDtypeStruct((b, 1), jnp.float32),
    )(x, w1, b1, wp, bp)


def kernel(x_categorical, tables, W1, b1, Wp, bp):
    f, v, d = tables.shape
    b = x_categorical.shape[0]
    h = W1.shape[1]
    n_rows = b * f
    flat_idx = x_categorical + (jnp.arange(f, dtype=jnp.int32) * v)[None, :]
    idx3 = flat_idx.reshape(_NW, (n_rows // _NW) // _IDXS_PER_DMA, _IDXS_PER_DMA)
    table_flat = tables.reshape(f * v, d)
    emb = _sc_gather(table_flat, idx3, n_rows, d)
    x = emb.reshape(b, f * d)
    return _tc_mlp(x, W1, b1.reshape(1, h), Wp, bp.reshape(1, 1), 2048)
